# baseline (speedup 1.0000x reference)
import functools

import jax
import jax.numpy as jnp
from jax import lax
from jax.experimental import pallas as pl
from jax.experimental.pallas import tpu as pltpu
from jax.experimental.pallas import tpu_sc as plsc

N = 10000
E = 320000
C = 128
NL = 48

NC = 2
NS = 16
NW = NC * NS
K = 128
WPT = 80
BLK = 16
NBLK = WPT // BLK
NCHUNK = NW * WPT
E_PAD = NCHUNK * K
NP = 10240
ROWS_PER_SUB = NP // NS

_mesh = plsc.VectorSubcoreMesh(core_axis_name="c", subcore_axis_name="s")


@functools.partial(
    pl.kernel,
    mesh=_mesh,
    out_type=jax.ShapeDtypeStruct((NC * NP, C), jnp.float32),
    scratch_types=[
        pltpu.VMEM((2, BLK, K), jnp.int32),
        pltpu.VMEM((2, BLK, K), jnp.int32),
        pltpu.VMEM((2, BLK, K), jnp.float32),
        pltpu.VMEM((K, C), jnp.float32),
        pltpu.VMEM((K, C), jnp.float32),
        pltpu.VMEM_SHARED((NP, C), jnp.float32),
        pltpu.SemaphoreType.DMA,
        pltpu.SemaphoreType.DMA,
        pltpu.SemaphoreType.DMA,
        pltpu.SemaphoreType.DMA,
        pltpu.SemaphoreType.DMA,
        pltpu.SemaphoreType.DMA,
    ],
)
def _sc_edge_pass(h_hbm, src_hbm, dst_hbm, w_hbm, out_hbm,
                  srcb, dstb, wb, rows_a, rows_b, agg_sh,
                  semi0, semi1, sema, semb, semsa, semsb):
    c = lax.axis_index("c")
    s = lax.axis_index("s")
    wid = s * NC + c
    semi = (semi0, semi1)

    def _idx_block_start(b, st):
        pltpu.async_copy(src_hbm.at[wid, b], srcb.at[st], semi[st])
        pltpu.async_copy(dst_hbm.at[wid, b], dstb.at[st], semi[st])
        pltpu.async_copy(w_hbm.at[wid, b], wb.at[st], semi[st])

    def _idx_block_wait(b, st):
        pltpu.make_async_copy(src_hbm.at[wid, b], srcb.at[st], semi[st]).wait()
        pltpu.make_async_copy(dst_hbm.at[wid, b], dstb.at[st], semi[st]).wait()
        pltpu.make_async_copy(w_hbm.at[wid, b], wb.at[st], semi[st]).wait()

    _idx_block_start(0, 0)

    def _zero_row(i, carry):
        for j in range(C // 16):
            rows_a[i, pl.ds(j * 16, 16)] = jnp.zeros((16,), jnp.float32)
        return carry
    lax.fori_loop(0, K, _zero_row, 0)
    for k in range(ROWS_PER_SUB // K):
        pltpu.sync_copy(rows_a,
                        agg_sh.at[pl.ds(s * ROWS_PER_SUB + k * K, K)])
    plsc.subcore_barrier()

    def _scale(rows, wset, t):
        def _group(g, cc):
            w16 = wset[t, pl.ds(g * 16, 16)]
            for l in range(16):
                wvec = lax.broadcast_in_dim(w16[l], (16,), ())
                i = g * 16 + l
                for j in range(C // 16):
                    sl = pl.ds(j * 16, 16)
                    rows[i, sl] = rows[i, sl] * wvec
            return cc
        lax.fori_loop(0, K // 16, _group, 0)

    _idx_block_wait(0, 0)
    pltpu.async_copy(h_hbm.at[srcb.at[0, 0]], rows_a, sema)

    for b in range(NBLK):
        st = b % 2
        if b + 1 < NBLK:
            _idx_block_start(b + 1, 1 - st)
        src_s, dst_s, w_s = srcb.at[st], dstb.at[st], wb.at[st]

        def _pair(hh, carry):
            t0 = hh * 2
            t1 = t0 + 1
            pltpu.make_async_copy(h_hbm.at[src_s.at[t0]], rows_a, sema).wait()

            @pl.when(hh > 0)
            def _():
                pltpu.make_async_copy(rows_b, agg_sh.at[dst_s.at[t1]],
                                      semsb).wait()
            pltpu.async_copy(h_hbm.at[src_s.at[t1]], rows_b, semb)
            _scale(rows_a, w_s, t0)
            pltpu.async_copy(rows_a, agg_sh.at[dst_s.at[t0]], semsa, add=True)
            pltpu.make_async_copy(h_hbm.at[src_s.at[t1]], rows_b, semb).wait()
            pltpu.make_async_copy(rows_a, agg_sh.at[dst_s.at[t0]], semsa).wait()

            @pl.when(hh < BLK // 2 - 1)
            def _():
                pltpu.async_copy(h_hbm.at[src_s.at[t0 + 2]], rows_a, sema)
            _scale(rows_b, w_s, t1)
            pltpu.async_copy(rows_b, agg_sh.at[dst_s.at[t1]], semsb, add=True)
            return carry

        lax.fori_loop(0, BLK // 2, _pair, 0)
        pltpu.make_async_copy(rows_b, agg_sh.at[dst_s.at[BLK - 1]], semsb).wait()
        if b + 1 < NBLK:
            _idx_block_wait(b + 1, 1 - st)
            pltpu.async_copy(h_hbm.at[srcb.at[1 - st, 0]], rows_a, sema)

    plsc.subcore_barrier()

    for k in range(ROWS_PER_SUB // K):
        r0 = s * ROWS_PER_SUB + k * K
        pltpu.sync_copy(agg_sh.at[pl.ds(r0, K)],
                        out_hbm.at[pl.ds(c * NP + r0, K)])


_BR = 1000


def _tc_pre(x, W1, W2):
    def body(x_ref, w1_ref, w2_ref, h_ref, skip_ref):
        xb = x_ref[...]
        h_ref[...] = jnp.dot(xb, w1_ref[...], preferred_element_type=jnp.float32)
        skip_ref[...] = jnp.dot(xb, w2_ref[...], preferred_element_type=jnp.float32)

    return pl.pallas_call(
        body,
        grid=(N // _BR,),
        in_specs=[
            pl.BlockSpec((_BR, C), lambda i: (i, 0)),
            pl.BlockSpec((C, C), lambda i: (0, 0)),
            pl.BlockSpec((C, C), lambda i: (0, 0)),
        ],
        out_specs=[
            pl.BlockSpec((_BR, C), lambda i: (i, 0)),
            pl.BlockSpec((_BR, C), lambda i: (i, 0)),
        ],
        out_shape=[
            jax.ShapeDtypeStruct((N, C), jnp.float32),
            jax.ShapeDtypeStruct((N, C), jnp.float32),
        ],
    )(x, W1, W2)


def _tc_mid(p0, p1, skip, b, W1, W2):
    def body(p0_ref, p1_ref, skip_ref, b_ref, w1_ref, w2_ref, h_ref, skip2_ref):
        t = jax.nn.relu(p0_ref[...] + p1_ref[...] + skip_ref[...] + b_ref[...])
        h_ref[...] = jnp.dot(t, w1_ref[...], preferred_element_type=jnp.float32)
        skip2_ref[...] = jnp.dot(t, w2_ref[...], preferred_element_type=jnp.float32)

    return pl.pallas_call(
        body,
        grid=(N // _BR,),
        in_specs=[
            pl.BlockSpec((_BR, C), lambda i: (i, 0)),
            pl.BlockSpec((_BR, C), lambda i: (i, 0)),
            pl.BlockSpec((_BR, C), lambda i: (i, 0)),
            pl.BlockSpec((1, C), lambda i: (0, 0)),
            pl.BlockSpec((C, C), lambda i: (0, 0)),
            pl.BlockSpec((C, C), lambda i: (0, 0)),
        ],
        out_specs=[
            pl.BlockSpec((_BR, C), lambda i: (i, 0)),
            pl.BlockSpec((_BR, C), lambda i: (i, 0)),
        ],
        out_shape=[
            jax.ShapeDtypeStruct((N, C), jnp.float32),
            jax.ShapeDtypeStruct((N, C), jnp.float32),
        ],
    )(p0, p1, skip, b, W1, W2)


def _tc_final(p0, p1, skip, b, Wd, bd):
    def body(p0_ref, p1_ref, skip_ref, b_ref, wd_ref, bd_ref, o_ref):
        t = jax.nn.relu(p0_ref[...] + p1_ref[...] + skip_ref[...] + b_ref[...])
        o_ref[...] = jnp.dot(t, wd_ref[...], preferred_element_type=jnp.float32) + bd_ref[...]

    return pl.pallas_call(
        body,
        grid=(N // _BR,),
        in_specs=[
            pl.BlockSpec((_BR, C), lambda i: (i, 0)),
            pl.BlockSpec((_BR, C), lambda i: (i, 0)),
            pl.BlockSpec((_BR, C), lambda i: (i, 0)),
            pl.BlockSpec((1, C), lambda i: (0, 0)),
            pl.BlockSpec((C, NL), lambda i: (0, 0)),
            pl.BlockSpec((1, NL), lambda i: (0, 0)),
        ],
        out_specs=pl.BlockSpec((_BR, NL), lambda i: (i, 0)),
        out_shape=jax.ShapeDtypeStruct((N, NL), jnp.float32),
    )(p0, p1, skip, b, Wd, bd)


def kernel(x, edge_index, edge_weight, W1_1, W2_1, b1, W1_2, W2_2, b2, Wd, bd):
    npad = E_PAD - E
    pad_idx = (jnp.arange(npad, dtype=jnp.int32) * 13) % N
    src = jnp.concatenate([edge_index[0].astype(jnp.int32), pad_idx]).reshape(NW, NBLK, BLK, K)
    dst = jnp.concatenate([edge_index[1].astype(jnp.int32), pad_idx]).reshape(NW, NBLK, BLK, K)
    w = jnp.concatenate([edge_weight.astype(jnp.float32),
                         jnp.zeros((npad,), jnp.float32)]).reshape(NW, NBLK, BLK, K)
    b1r = b1.reshape(1, C)
    b2r = b2.reshape(1, C)
    bdr = bd.reshape(1, NL)

    h, skip = _tc_pre(x, W1_1, W2_1)
    p = _sc_edge_pass(h, src, dst, w)
    h, skip = _tc_mid(p[:N], p[NP:NP + N], skip, b1r, W1_2, W2_2)
    p = _sc_edge_pass(h, src, dst, w)
    h, skip = _tc_mid(p[:N], p[NP:NP + N], skip, b2r, W1_2, W2_2)
    p = _sc_edge_pass(h, src, dst, w)
    return _tc_final(p[:N], p[NP:NP + N], skip, b2r, Wd, bdr)


# speedup vs baseline: 1.0852x; 1.0852x over previous
import functools

import numpy as np

import jax
import jax.numpy as jnp
from jax import lax
from jax.experimental.layout import Format, Layout, with_layout_constraint
from jax.experimental import pallas as pl
from jax.experimental.pallas import tpu as pltpu
from jax.experimental.pallas import tpu_sc as plsc

N = 10000
E = 320000
C = 128
NL = 48

NC = 2
NS = 16
NW = NC * NS
K = 64
WPT = 160
BLK = 16
NBLK = WPT // BLK
NCHUNK = NW * WPT
E_PAD = NCHUNK * K
NP = 10240
ROWS_PER_SUB = NP // NS

_ww = np.arange(C // 2)
_COLS_LO = (32 * (_ww // 16) + _ww % 16).tolist()
_COLS_HI = (32 * (_ww // 16) + _ww % 16 + 16).tolist()

_mesh = plsc.VectorSubcoreMesh(core_axis_name="c", subcore_axis_name="s")


@functools.partial(
    pl.kernel,
    mesh=_mesh,
    out_type=jax.ShapeDtypeStruct((NC * NP, C), jnp.float32),
    scratch_types=[
        pltpu.VMEM((2, BLK, K), jnp.int32),
        pltpu.VMEM((2, BLK, K), jnp.int32),
        pltpu.VMEM((2, BLK, K), jnp.float32),
        pltpu.VMEM((K, C // 2), jnp.int32),
        pltpu.VMEM((K, C // 2), jnp.int32),
        pltpu.VMEM((K, C), jnp.float32),
        pltpu.VMEM((K, C), jnp.float32),
        pltpu.VMEM_SHARED((NP, C), jnp.float32),
        pltpu.SemaphoreType.DMA,
        pltpu.SemaphoreType.DMA,
        pltpu.SemaphoreType.DMA,
        pltpu.SemaphoreType.DMA,
        pltpu.SemaphoreType.DMA,
        pltpu.SemaphoreType.DMA,
    ],
    compiler_params=pltpu.CompilerParams(needs_layout_passes=False),
)
def _sc_edge_pass(h_hbm, src_hbm, dst_hbm, w_hbm, out_hbm,
                  srcb, dstb, wb, g16a, g16b, rfa, rfb, agg_sh,
                  semi0, semi1, sema, semb, semsa, semsb):
    c = lax.axis_index("c")
    s = lax.axis_index("s")
    wid = s * NC + c
    semi = (semi0, semi1)

    def _idx_block_start(b, st):
        pltpu.async_copy(src_hbm.at[wid, b], srcb.at[st], semi[st])
        pltpu.async_copy(dst_hbm.at[wid, b], dstb.at[st], semi[st])
        pltpu.async_copy(w_hbm.at[wid, b], wb.at[st], semi[st])

    def _idx_block_wait(b, st):
        pltpu.make_async_copy(src_hbm.at[wid, b], srcb.at[st], semi[st]).wait()
        pltpu.make_async_copy(dst_hbm.at[wid, b], dstb.at[st], semi[st]).wait()
        pltpu.make_async_copy(w_hbm.at[wid, b], wb.at[st], semi[st]).wait()

    _idx_block_start(0, 0)

    def _zero_row(i, carry):
        for j in range(C // 16):
            rfa[i, pl.ds(j * 16, 16)] = jnp.zeros((16,), jnp.float32)
        return carry
    lax.fori_loop(0, K, _zero_row, 0)
    for k in range(ROWS_PER_SUB // K):
        pltpu.sync_copy(rfa, agg_sh.at[pl.ds(s * ROWS_PER_SUB + k * K, K)])
    plsc.subcore_barrier()

    def _scale(r16, rf, wset, t):
        def _group(gg, cc):
            w16 = wset[t, pl.ds(gg * 16, 16)]
            for l in range(16):
                wvec = lax.broadcast_in_dim(w16[l], (16,), ())
                i = gg * 16 + l
                for g in range(C // 32):
                    r32 = r16[i, pl.ds(g * 16, 16)]
                    r = plsc.bitcast(r32, jnp.bfloat16)
                    a, b2 = plsc.unpack(r, format=plsc.PackFormat.INTERLEAVED)
                    rf[i, pl.ds(g * 32, 16)] = a * wvec
                    rf[i, pl.ds(g * 32 + 16, 16)] = b2 * wvec
            return cc
        lax.fori_loop(0, K // 16, _group, 0)

    _idx_block_wait(0, 0)

    def _run_block(b, st):
        @pl.when(b + 1 < NBLK)
        def _():
            _idx_block_start(b + 1, 1 - st)
        src_s, dst_s, w_s = srcb.at[st], dstb.at[st], wb.at[st]
        pltpu.async_copy(h_hbm.at[src_s.at[0]], g16a, sema)
        pltpu.async_copy(h_hbm.at[src_s.at[1]], g16b, semb)

        def _pair(hh, carry):
            t0 = hh * 2
            t1 = t0 + 1
            pltpu.make_async_copy(h_hbm.at[src_s.at[t0]], g16a, sema).wait()

            @pl.when(hh > 0)
            def _():
                pltpu.make_async_copy(rfa, agg_sh.at[dst_s.at[t0]],
                                      semsa).wait()
            _scale(g16a, rfa, w_s, t0)

            @pl.when(hh < BLK // 2 - 1)
            def _():
                pltpu.async_copy(h_hbm.at[src_s.at[t0 + 2]], g16a, sema)
            pltpu.async_copy(rfa, agg_sh.at[dst_s.at[t0]], semsa, add=True)

            pltpu.make_async_copy(h_hbm.at[src_s.at[t1]], g16b, semb).wait()

            @pl.when(hh > 0)
            def _():
                pltpu.make_async_copy(rfb, agg_sh.at[dst_s.at[t1]],
                                      semsb).wait()
            _scale(g16b, rfb, w_s, t1)

            @pl.when(hh < BLK // 2 - 1)
            def _():
                pltpu.async_copy(h_hbm.at[src_s.at[t1 + 2]], g16b, semb)
            pltpu.async_copy(rfb, agg_sh.at[dst_s.at[t1]], semsb, add=True)
            return carry

        lax.fori_loop(0, BLK // 2, _pair, 0)
        pltpu.make_async_copy(rfa, agg_sh.at[dst_s.at[BLK - 2]], semsa).wait()
        pltpu.make_async_copy(rfb, agg_sh.at[dst_s.at[BLK - 1]], semsb).wait()

        @pl.when(b + 1 < NBLK)
        def _():
            _idx_block_wait(b + 1, 1 - st)

    def _block(b, carry):
        par = lax.rem(b, 2)

        @pl.when(par == 0)
        def _():
            _run_block(b, 0)

        @pl.when(par == 1)
        def _():
            _run_block(b, 1)
        return carry

    lax.fori_loop(0, NBLK, _block, 0)
    plsc.subcore_barrier()

    for k in range(ROWS_PER_SUB // K):
        r0 = s * ROWS_PER_SUB + k * K
        pltpu.sync_copy(agg_sh.at[pl.ds(r0, K)],
                        out_hbm.at[pl.ds(c * NP + r0, K)])


_BR = 1000


def _pack_h(t, w1a_ref, w1b_ref):
    lo = jnp.dot(t, w1a_ref[...], preferred_element_type=jnp.float32)
    hi = jnp.dot(t, w1b_ref[...], preferred_element_type=jnp.float32)
    lo16 = lax.bitcast_convert_type(lo.astype(jnp.bfloat16), jnp.uint16)
    hi16 = lax.bitcast_convert_type(hi.astype(jnp.bfloat16), jnp.uint16)
    word = lo16.astype(jnp.uint32) | (hi16.astype(jnp.uint32) << 16)
    return lax.bitcast_convert_type(word, jnp.int32)


def _tc_pre(x, W1a, W1b, W2):
    def body(x_ref, w1a_ref, w1b_ref, w2_ref, h_ref, skip_ref):
        xb = x_ref[...]
        h_ref[...] = _pack_h(xb, w1a_ref, w1b_ref)
        skip_ref[...] = jnp.dot(xb, w2_ref[...], preferred_element_type=jnp.float32)

    return pl.pallas_call(
        body,
        grid=(N // _BR,),
        in_specs=[
            pl.BlockSpec((_BR, C), lambda i: (i, 0)),
            pl.BlockSpec((C, C // 2), lambda i: (0, 0)),
            pl.BlockSpec((C, C // 2), lambda i: (0, 0)),
            pl.BlockSpec((C, C), lambda i: (0, 0)),
        ],
        out_specs=[
            pl.BlockSpec((_BR, C // 2), lambda i: (i, 0)),
            pl.BlockSpec((_BR, C), lambda i: (i, 0)),
        ],
        out_shape=[
            jax.ShapeDtypeStruct((N, C // 2), jnp.int32),
            jax.ShapeDtypeStruct((N, C), jnp.float32),
        ],
    )(x, W1a, W1b, W2)


def _tc_mid(p0, p1, skip, b, W1a, W1b, W2):
    def body(p0_ref, p1_ref, skip_ref, b_ref, w1a_ref, w1b_ref, w2_ref,
             h_ref, skip2_ref):
        t = jax.nn.relu(p0_ref[...] + p1_ref[...] + skip_ref[...] + b_ref[...])
        h_ref[...] = _pack_h(t, w1a_ref, w1b_ref)
        skip2_ref[...] = jnp.dot(t, w2_ref[...], preferred_element_type=jnp.float32)

    return pl.pallas_call(
        body,
        grid=(N // _BR,),
        in_specs=[
            pl.BlockSpec((_BR, C), lambda i: (i, 0)),
            pl.BlockSpec((_BR, C), lambda i: (i, 0)),
            pl.BlockSpec((_BR, C), lambda i: (i, 0)),
            pl.BlockSpec((1, C), lambda i: (0, 0)),
            pl.BlockSpec((C, C // 2), lambda i: (0, 0)),
            pl.BlockSpec((C, C // 2), lambda i: (0, 0)),
            pl.BlockSpec((C, C), lambda i: (0, 0)),
        ],
        out_specs=[
            pl.BlockSpec((_BR, C // 2), lambda i: (i, 0)),
            pl.BlockSpec((_BR, C), lambda i: (i, 0)),
        ],
        out_shape=[
            jax.ShapeDtypeStruct((N, C // 2), jnp.int32),
            jax.ShapeDtypeStruct((N, C), jnp.float32),
        ],
    )(p0, p1, skip, b, W1a, W1b, W2)


def _tc_final(p0, p1, skip, b, Wd, bd):
    def body(p0_ref, p1_ref, skip_ref, b_ref, wd_ref, bd_ref, o_ref):
        t = jax.nn.relu(p0_ref[...] + p1_ref[...] + skip_ref[...] + b_ref[...])
        o_ref[...] = jnp.dot(t, wd_ref[...], preferred_element_type=jnp.float32) + bd_ref[...]

    return pl.pallas_call(
        body,
        grid=(N // _BR,),
        in_specs=[
            pl.BlockSpec((_BR, C), lambda i: (i, 0)),
            pl.BlockSpec((_BR, C), lambda i: (i, 0)),
            pl.BlockSpec((_BR, C), lambda i: (i, 0)),
            pl.BlockSpec((1, C), lambda i: (0, 0)),
            pl.BlockSpec((C, NL), lambda i: (0, 0)),
            pl.BlockSpec((1, NL), lambda i: (0, 0)),
        ],
        out_specs=pl.BlockSpec((_BR, NL), lambda i: (i, 0)),
        out_shape=jax.ShapeDtypeStruct((N, NL), jnp.float32),
    )(p0, p1, skip, b, Wd, bd)


def kernel(x, edge_index, edge_weight, W1_1, W2_1, b1, W1_2, W2_2, b2, Wd, bd):
    npad = E_PAD - E
    pad_idx = (jnp.arange(npad, dtype=jnp.int32) * 13) % N
    src = jnp.concatenate([edge_index[0].astype(jnp.int32), pad_idx]).reshape(NW, NBLK, BLK, K)
    dst = jnp.concatenate([edge_index[1].astype(jnp.int32), pad_idx]).reshape(NW, NBLK, BLK, K)
    w = jnp.concatenate([edge_weight.astype(jnp.float32),
                         jnp.zeros((npad,), jnp.float32)]).reshape(NW, NBLK, BLK, K)
    cols_lo = jnp.asarray(_COLS_LO, dtype=jnp.int32)
    cols_hi = jnp.asarray(_COLS_HI, dtype=jnp.int32)
    W1a_1, W1b_1 = W1_1[:, cols_lo], W1_1[:, cols_hi]
    W1a_2, W1b_2 = W1_2[:, cols_lo], W1_2[:, cols_hi]
    b1r = b1.reshape(1, C)
    b2r = b2.reshape(1, C)
    bdr = bd.reshape(1, NL)

    _hfmt = Layout(major_to_minor=(0, 1), tiling=((16,),))

    h, skip = _tc_pre(x, W1a_1, W1b_1, W2_1)
    p = _sc_edge_pass(with_layout_constraint(h, _hfmt), src, dst, w)
    h, skip = _tc_mid(p[:N], p[NP:NP + N], skip, b1r, W1a_2, W1b_2, W2_2)
    p = _sc_edge_pass(with_layout_constraint(h, _hfmt), src, dst, w)
    h, skip = _tc_mid(p[:N], p[NP:NP + N], skip, b2r, W1a_2, W1b_2, W2_2)
    p = _sc_edge_pass(with_layout_constraint(h, _hfmt), src, dst, w)
    return _tc_final(p[:N], p[NP:NP + N], skip, b2r, Wd, bdr)
